# baseline (speedup 1.0000x reference)
import jax
import jax.numpy as jnp
from jax import lax
from jax.experimental import pallas as pl


_R = 64


def _tc_body(be_ref, bo_ref, oe_ref, oo_ref, w_ref, out_ref):
    r, h = be_ref.shape
    ids_e = be_ref[...] * 2 + oe_ref[...]
    ids_o = bo_ref[...] * 2 + oo_ref[...]
    w = w_ref[...]
    tok = lax.broadcasted_iota(jnp.int32, (6, h), 0)
    for g in range(r):
        row_e = jnp.broadcast_to(ids_e[g:g + 1, :], (6, h))
        row_o = jnp.broadcast_to(ids_o[g:g + 1, :], (6, h))
        oh_e = (row_e == tok).astype(jnp.float32)
        oh_o = (row_o == tok).astype(jnp.float32)
        onehot = jnp.concatenate([oh_e, oh_o], axis=0)
        res = lax.dot_general(onehot, w, (((0,), (0,)), ((), ())),
                              preferred_element_type=jnp.float32)
        out_ref[pl.ds(g * h, h), :] = res


def kernel(basis, outcome, table):
    n, c = basis.shape
    h = c // 2
    basis_e = basis[:, 0::2]
    basis_o = basis[:, 1::2]
    outcome_e = outcome[:, 0::2]
    outcome_o = outcome[:, 1::2]
    w = jnp.zeros((12, 128), jnp.float32)
    w = w.at[0:6, 0:64].set(table).at[6:12, 64:128].set(table)

    grid = (n // _R,)
    out = pl.pallas_call(
        _tc_body,
        grid=grid,
        in_specs=[
            pl.BlockSpec((_R, h), lambda i: (i, 0)),
            pl.BlockSpec((_R, h), lambda i: (i, 0)),
            pl.BlockSpec((_R, h), lambda i: (i, 0)),
            pl.BlockSpec((_R, h), lambda i: (i, 0)),
            pl.BlockSpec((12, 128), lambda i: (0, 0)),
        ],
        out_specs=pl.BlockSpec((_R * h, 128), lambda i: (i, 0)),
        out_shape=jax.ShapeDtypeStruct((n * h, 128), jnp.float32),
    )(basis_e, basis_o, outcome_e, outcome_o, w)
    return out.reshape(n, c, 64)
